# Initial kernel scaffold; baseline (speedup 1.0000x reference)
import jax
import jax.numpy as jnp
from jax.experimental import pallas as pl


def kernel(x, emb):
    raise NotImplementedError("write your pallas kernel here")


# speedup vs baseline: 6.3391x; 6.3391x over previous
import jax
import jax.numpy as jnp
from jax import lax
from jax.experimental import pallas as pl
from jax.experimental.pallas import tpu as pltpu
from jax.experimental.pallas import tpu_sc as plsc

_NC, _NS = 2, 16
_NW = _NC * _NS
_D = 128
_B = 4096 * 200
_ROWS = _B // _D
_RPW = _ROWS // _NW


def _body(x_hbm, emb_hbm, out_hbm, idx_v, rows_v, sem):
    wid = lax.axis_index("s") * _NC + lax.axis_index("c")
    r0 = wid * _RPW
    pltpu.sync_copy(x_hbm.at[pl.ds(r0, _RPW)], idx_v)

    def step(j, carry):
        pltpu.async_copy(emb_hbm.at[idx_v.at[j]], rows_v, sem).wait()
        pltpu.sync_copy(rows_v, out_hbm.at[pl.ds((r0 + j) * _D, _D)])
        return carry

    lax.fori_loop(0, _RPW, step, 0)


def kernel(x, emb):
    xf = x.reshape(_ROWS, _D)
    mesh = plsc.VectorSubcoreMesh(core_axis_name="c", subcore_axis_name="s")
    out = pl.kernel(
        _body,
        out_type=jax.ShapeDtypeStruct((_B, _D), jnp.float32),
        mesh=mesh,
        scratch_types=[
            pltpu.VMEM((_RPW, _D), jnp.int32),
            pltpu.VMEM((_D, _D), jnp.float32),
            pltpu.SemaphoreType.DMA,
        ],
    )(xf, emb)
    return out.reshape(x.shape[0], x.shape[1], _D)


# speedup vs baseline: 9.1384x; 1.4416x over previous
import jax
import jax.numpy as jnp
from jax import lax
from jax.experimental import pallas as pl
from jax.experimental.pallas import tpu as pltpu
from jax.experimental.pallas import tpu_sc as plsc

_NC, _NS = 2, 16
_NW = _NC * _NS
_D = 128
_B = 4096 * 200
_ROWS = _B // _D
_RPW = _ROWS // _NW


_NB = 4


def _body(x_hbm, emb_hbm, out_hbm, idx_v, b0, b1, b2, b3, s0, s1, s2, s3):
    bufs = (b0, b1, b2, b3)
    sems = (s0, s1, s2, s3)
    wid = lax.axis_index("s") * _NC + lax.axis_index("c")
    r0 = wid * _RPW
    pltpu.sync_copy(x_hbm.at[pl.ds(r0, _RPW)], idx_v)
    for b in range(_NB):
        pltpu.async_copy(emb_hbm.at[idx_v.at[b]], bufs[b], sems[b])

    def outer(i, carry):
        j = i * _NB
        for b in range(_NB):
            t = j + b
            pltpu.make_async_copy(emb_hbm.at[idx_v.at[t]], bufs[b], sems[b]).wait()
            pltpu.sync_copy(bufs[b], out_hbm.at[pl.ds((r0 + t) * _D, _D)])
            tn = jnp.minimum(t + _NB, _RPW - 1)
            pltpu.async_copy(emb_hbm.at[idx_v.at[tn]], bufs[b], sems[b])
        return carry

    lax.fori_loop(0, _RPW // _NB, outer, 0)
    for b in range(_NB):
        pltpu.make_async_copy(emb_hbm.at[idx_v.at[0]], bufs[b], sems[b]).wait()


def kernel(x, emb):
    xf = x.reshape(_ROWS, _D)
    mesh = plsc.VectorSubcoreMesh(core_axis_name="c", subcore_axis_name="s")
    out = pl.kernel(
        _body,
        out_type=jax.ShapeDtypeStruct((_B, _D), jnp.float32),
        mesh=mesh,
        scratch_types=[
            pltpu.VMEM((_RPW, _D), jnp.int32),
        ] + [pltpu.VMEM((_D, _D), jnp.float32)] * _NB
          + [pltpu.SemaphoreType.DMA] * _NB,
    )(xf, emb)
    return out.reshape(x.shape[0], x.shape[1], _D)
